# baseline (speedup 1.0000x reference)
import functools

import jax
import jax.numpy as jnp
import numpy as np
from jax import lax
from jax.experimental import pallas as pl
from jax.experimental.pallas import tpu as pltpu
from jax.experimental.pallas import tpu_sc as plsc

_N, _E, _D, _H, _DH, _HID, _G, _OUT = 10000, 320000, 128, 8, 16, 128, 64, 10
_NP = 10112
_NC, _NS = 2, 16
_NW = _NC * _NS
_CH = 64
_NBUF = 3
_GCH = 27
_NG = 6
_CPW = _GCH * _NG
_EP = _NW * _CPW * _CH
_TPR = _NP // _NS

_PERM = np.array([h * _DH + d for d in range(_DH) for h in range(_H)], dtype=np.int32)


def _build_att(a):
  eye = jnp.eye(_H, dtype=a.dtype)
  A = jnp.einsum("hd,hj->dhj", a, eye)
  A = jnp.concatenate([A, A], axis=-1)
  return A.reshape(_HID, 2 * _H)


def _tile8(a):
  return jnp.concatenate([a] * 8, axis=1)


def _head_body(x_ref, w_ref, as_ref, ad_ref, ht_ref, s_ref, d_ref, ws_ref):
  ht = jnp.dot(x_ref[...], w_ref[...], preferred_element_type=jnp.float32)
  ht_ref[...] = ht
  s = jnp.dot(ht, as_ref[...], preferred_element_type=jnp.float32)
  d = jnp.dot(ht, ad_ref[...], preferred_element_type=jnp.float32)
  s_ref[...] = s
  d_ref[...] = d
  e = s + d
  ws_ref[...] = jnp.exp(jnp.where(e >= 0, e, 0.2 * e))


_BR = 1264
_NB = _NP // _BR

_row_spec = pl.BlockSpec((_BR, _HID), lambda i: (i, 0))
_nar_spec = pl.BlockSpec((_BR, 16), lambda i: (i, 0))
_full = lambda shape: pl.BlockSpec(shape, lambda i: tuple(0 for _ in shape))

_head = pl.pallas_call(
    _head_body,
    grid=(_NB,),
    in_specs=[_row_spec, _full((_HID, _HID)), _full((_HID, 16)), _full((_HID, 16))],
    out_specs=[_row_spec, _nar_spec, _nar_spec, _nar_spec],
    out_shape=[
        jax.ShapeDtypeStruct((_NP, _HID), jnp.float32),
        jax.ShapeDtypeStruct((_NP, 16), jnp.float32),
        jax.ShapeDtypeStruct((_NP, 16), jnp.float32),
        jax.ShapeDtypeStruct((_NP, 16), jnp.float32),
    ],
)


def _combine(acc_ref, dacc_ref, ht_ref, ws_ref, b_ref):
  wsf = ws_ref[...]
  num = acc_ref[0] + acc_ref[1] + ht_ref[...] * _tile8(wsf)
  den = _tile8(dacc_ref[0] + dacc_ref[1] + wsf) + 1e-16
  return jnp.maximum(num / den + b_ref[...], 0.0)


def _mid_body(acc_ref, dacc_ref, ht_ref, ws_ref, b_ref, w_ref, as_ref, ad_ref,
              ht2_ref, s_ref, d_ref, ws2_ref):
  y = _combine(acc_ref, dacc_ref, ht_ref, ws_ref, b_ref)
  ht2 = jnp.dot(y, w_ref[...], preferred_element_type=jnp.float32)
  ht2_ref[...] = ht2
  s = jnp.dot(ht2, as_ref[...], preferred_element_type=jnp.float32)
  d = jnp.dot(ht2, ad_ref[...], preferred_element_type=jnp.float32)
  s_ref[...] = s
  d_ref[...] = d
  e = s + d
  ws2_ref[...] = jnp.exp(jnp.where(e >= 0, e, 0.2 * e))


_acc_spec = pl.BlockSpec((_NC, _BR, _HID), lambda i: (0, i, 0))
_dacc_spec = pl.BlockSpec((_NC, _BR, 16), lambda i: (0, i, 0))

_mid = pl.pallas_call(
    _mid_body,
    grid=(_NB,),
    in_specs=[_acc_spec, _dacc_spec, _row_spec, _nar_spec, _full((1, _HID)),
              _full((_HID, _HID)), _full((_HID, 16)), _full((_HID, 16))],
    out_specs=[_row_spec, _nar_spec, _nar_spec, _nar_spec],
    out_shape=[
        jax.ShapeDtypeStruct((_NP, _HID), jnp.float32),
        jax.ShapeDtypeStruct((_NP, 16), jnp.float32),
        jax.ShapeDtypeStruct((_NP, 16), jnp.float32),
        jax.ShapeDtypeStruct((_NP, 16), jnp.float32),
    ],
)


def _final_body(acc_ref, dacc_ref, ht_ref, ws_ref, b_ref, batch_ref,
                wc1_ref, bc1_ref, wc2_ref, bc2_ref, wc3_ref, bc3_ref, out_ref,
                s_acc, c_acc):
  i = pl.program_id(0)
  h2 = _combine(acc_ref, dacc_ref, ht_ref, ws_ref, b_ref)
  bt = batch_ref[0]
  Pt = (lax.broadcasted_iota(jnp.int32, (_G, _BR), 0) == bt).astype(jnp.float32)
  s = jnp.dot(Pt, h2, preferred_element_type=jnp.float32)
  cnt = jnp.dot(Pt, jnp.ones((_BR, _HID), jnp.float32),
                preferred_element_type=jnp.float32)

  @pl.when(i == 0)
  def _():
    s_acc[...] = s
    c_acc[...] = cnt

  @pl.when(i > 0)
  def _():
    s_acc[...] = s_acc[...] + s
    c_acc[...] = c_acc[...] + cnt

  @pl.when(i == _NB - 1)
  def _():
    g = s_acc[...] / jnp.maximum(c_acc[...], 1.0)
    z = jnp.maximum(jnp.dot(g, wc1_ref[...], preferred_element_type=jnp.float32)
                    + bc1_ref[...], 0.0)
    z = jnp.maximum(jnp.dot(z, wc2_ref[...], preferred_element_type=jnp.float32)
                    + bc2_ref[...], 0.0)
    out_ref[...] = jnp.dot(z, wc3_ref[...], preferred_element_type=jnp.float32) + bc3_ref[...]


_final = pl.pallas_call(
    _final_body,
    grid=(_NB,),
    in_specs=[_acc_spec, _dacc_spec, _row_spec, _nar_spec, _full((1, _HID)),
              pl.BlockSpec((1, 1, _BR), lambda i: (i, 0, 0)),
              _full((_HID, _DH)), _full((1, _DH)), _full((_DH, _DH // 2)),
              _full((1, _DH // 2)), _full((_DH // 2, _OUT)), _full((1, _OUT))],
    out_specs=pl.BlockSpec((_G, _OUT), lambda i: (0, 0)),
    out_shape=jax.ShapeDtypeStruct((_G, _OUT), jnp.float32),
    scratch_shapes=[pltpu.VMEM((_G, _HID), jnp.float32),
                    pltpu.VMEM((_G, _HID), jnp.float32)],
)


def _edge_body(src_hbm, dst_hbm, asrc_hbm, adst_hbm, ht_hbm, acc_out, dacc_out,
               acc_sp, dacc_sp, siv, div, gs, gd, gh,
               semg0, semg1, semg2, semsc0, semsc1, semsc2):
  core = lax.axis_index("c")
  sub = lax.axis_index("s")
  w = sub * _NC + core
  semg = (semg0, semg1, semg2)
  semsc = (semsc0, semsc1, semsc2)

  zero16 = jnp.zeros((16,), jnp.float32)

  def _z1(i, carry):
    for j in range(8):
      gh[i, pl.ds(j * 16, 16)] = zero16
    gs[i, :] = zero16
    return carry

  lax.fori_loop(0, _NBUF * _CH, _z1, 0)

  for off, ln in ((0, 192), (192, 192), (384, 192), (576, 56)):
    pltpu.sync_copy(gh.at[pl.ds(0, ln)], acc_sp.at[pl.ds(sub * _TPR + off, ln)])
    pltpu.sync_copy(gs.at[pl.ds(0, ln)], dacc_sp.at[pl.ds(sub * _TPR + off, ln)])
  plsc.subcore_barrier()

  def _fire_gathers(ci, b):
    return (
        pltpu.async_copy(asrc_hbm.at[siv.at[ci]], gs.at[pl.ds(b * _CH, _CH)], semg[b]),
        pltpu.async_copy(adst_hbm.at[div.at[ci]], gd.at[pl.ds(b * _CH, _CH)], semg[b]),
        pltpu.async_copy(ht_hbm.at[siv.at[ci]], gh.at[pl.ds(b * _CH, _CH)], semg[b]),
    )

  def _compute(b):
    base = b * _CH

    @plsc.parallel_loop(base, base + _CH, step=1, unroll=8)
    def _edge(c):
      e = gs[c, :] + gd[c, :]
      wv = jnp.exp(jnp.where(e >= 0, e, e * 0.2))
      gs[c, :] = wv
      for j2 in range(8):
        gh[c, pl.ds(j2 * 16, 16)] = gh[c, pl.ds(j2 * 16, 16)] * wv

  def _fire_scatters(ci, b):
    return (
        pltpu.async_copy(gh.at[pl.ds(b * _CH, _CH)], acc_sp.at[div.at[ci]], semsc[b], add=True),
        pltpu.async_copy(gs.at[pl.ds(b * _CH, _CH)], dacc_sp.at[div.at[ci]], semsc[b], add=True),
    )

  def _group(g, carry):
    row = w * _NG + g
    pltpu.sync_copy(src_hbm.at[row], siv)
    pltpu.sync_copy(dst_hbm.at[row], div)
    gcp = {0: _fire_gathers(0, 0)}
    scp = {}
    for ci in range(_GCH):
      b = ci % _NBUF
      if ci + 1 < _GCH:
        if ci - 2 >= 0:
          for cp in scp.pop(ci - 2):
            cp.wait()
        gcp[ci + 1] = _fire_gathers(ci + 1, (ci + 1) % _NBUF)
      for cp in gcp.pop(ci):
        cp.wait()
      _compute(b)
      scp[ci] = _fire_scatters(ci, b)
    for ci in (_GCH - 3, _GCH - 2, _GCH - 1):
      for cp in scp.pop(ci):
        cp.wait()
    return carry

  lax.fori_loop(0, _NG, _group, 0)
  plsc.subcore_barrier()

  pltpu.sync_copy(acc_sp.at[pl.ds(sub * _TPR, _TPR)],
                  acc_out.at[core, pl.ds(sub * _TPR, _TPR)])
  pltpu.sync_copy(dacc_sp.at[pl.ds(sub * _TPR, _TPR)],
                  dacc_out.at[core, pl.ds(sub * _TPR, _TPR)])


@functools.cache
def _make_edge_pass():
  return pl.kernel(
    _edge_body,
    out_type=[
        jax.ShapeDtypeStruct((_NC, _NP, _HID), jnp.float32),
        jax.ShapeDtypeStruct((_NC, _NP, 16), jnp.float32),
    ],
    mesh=plsc.VectorSubcoreMesh(core_axis_name="c", subcore_axis_name="s",
                                num_cores=_NC, num_subcores=_NS),
    compiler_params=pltpu.CompilerParams(use_tc_tiling_on_sc=False),
    scratch_types=[
        pltpu.VMEM_SHARED((_NP, _HID), jnp.float32),
        pltpu.VMEM_SHARED((_NP, 16), jnp.float32),
        pltpu.VMEM((_GCH, _CH), jnp.int32),
        pltpu.VMEM((_GCH, _CH), jnp.int32),
        pltpu.VMEM((_NBUF * _CH, 16), jnp.float32),
        pltpu.VMEM((_NBUF * _CH, 16), jnp.float32),
        pltpu.VMEM((_NBUF * _CH, _HID), jnp.float32),
        pltpu.SemaphoreType.DMA,
        pltpu.SemaphoreType.DMA,
        pltpu.SemaphoreType.DMA,
        pltpu.SemaphoreType.DMA,
        pltpu.SemaphoreType.DMA,
        pltpu.SemaphoreType.DMA,
    ],
  )


def _edge_pass(src3, dst3, asrc, adst, ht):
  return _make_edge_pass()(src3, dst3, asrc, adst, ht)


def kernel(x, edge_index, batch, W1, as1, ad1, b1, W2, as2, ad2, b2,
           Wc1, bc1, Wc2, bc2, Wc3, bc3):
  perm = _PERM
  xp = jnp.pad(x, ((0, _NP - _N), (0, 0)))
  pad = _N + (jnp.arange(_EP - _E, dtype=edge_index.dtype) % 16)
  src3 = jnp.concatenate([edge_index[0], pad]).reshape(_NW * _NG, _GCH, _CH)
  dst3 = jnp.concatenate([edge_index[1], pad]).reshape(_NW * _NG, _GCH, _CH)
  batch2 = jnp.pad(batch, (0, _NP - _N), constant_values=_G).reshape(_NB, 1, _BR)

  ht1, s1, d1, ws1 = _head(xp, W1[:, perm], _build_att(as1), _build_att(ad1))
  acc1, dacc1 = _edge_pass(src3, dst3, s1, d1, ht1)
  ht2, s2, d2, ws2 = _mid(acc1, dacc1, ht1, ws1, b1[perm].reshape(1, _HID),
                          W2[perm][:, perm], _build_att(as2), _build_att(ad2))
  acc2, dacc2 = _edge_pass(src3, dst3, s2, d2, ht2)
  return _final(acc2, dacc2, ht2, ws2, b2[perm].reshape(1, _HID), batch2,
                Wc1[perm], bc1.reshape(1, _DH), Wc2, bc2.reshape(1, _DH // 2),
                Wc3, bc3.reshape(1, _OUT))


# speedup vs baseline: 1.0808x; 1.0808x over previous
import functools

import jax
import jax.numpy as jnp
import numpy as np
from jax import lax
from jax.experimental import pallas as pl
from jax.experimental.pallas import tpu as pltpu
from jax.experimental.pallas import tpu_sc as plsc

_N, _E, _D, _H, _DH, _HID, _G, _OUT = 10000, 320000, 128, 8, 16, 128, 64, 10
_NP = 10112
_NC, _NS = 2, 16
_NW = _NC * _NS
_CH = 64
_NBUF = 3
_GCH = 27
_NG = 6
_CPW = _GCH * _NG
_EP = _NW * _CPW * _CH
_TPR = _NP // _NS

_PERM = np.array([h * _DH + d for d in range(_DH) for h in range(_H)], dtype=np.int32)


def _build_att(a):
  eye = jnp.eye(_H, dtype=a.dtype)
  A = jnp.einsum("hd,hj->dhj", a, eye)
  A = jnp.concatenate([A, A], axis=-1)
  return A.reshape(_HID, 2 * _H)


def _head_body(x_ref, w_ref, as_ref, ad_ref, ht_ref, s_ref, d_ref, ws_ref):
  ht = jnp.dot(x_ref[...], w_ref[...], preferred_element_type=jnp.float32)
  ht_ref[...] = ht
  s = jnp.dot(ht, as_ref[...], preferred_element_type=jnp.float32)
  d = jnp.dot(ht, ad_ref[...], preferred_element_type=jnp.float32)
  s_ref[...] = s
  d_ref[...] = d
  e = s + d
  ws_ref[...] = jnp.exp(jnp.where(e >= 0, e, 0.2 * e))


_BR = 1264
_NB = _NP // _BR

_row_spec = pl.BlockSpec((_BR, _HID), lambda i: (i, 0))
_nar_spec = pl.BlockSpec((_BR, 16), lambda i: (i, 0))
_full = lambda shape: pl.BlockSpec(shape, lambda i: tuple(0 for _ in shape))

_head = pl.pallas_call(
    _head_body,
    grid=(_NB,),
    in_specs=[_row_spec, _full((_HID, _HID)), _full((_HID, 16)), _full((_HID, 16))],
    out_specs=[_row_spec, _nar_spec, _nar_spec, _nar_spec],
    out_shape=[
        jax.ShapeDtypeStruct((_NP, _HID), jnp.float32),
        jax.ShapeDtypeStruct((_NP, 16), jnp.float32),
        jax.ShapeDtypeStruct((_NP, 16), jnp.float32),
        jax.ShapeDtypeStruct((_NP, 16), jnp.float32),
    ],
)


def _combine(acc_ref, dacc_ref, ht_ref, ws_ref, b_ref):
  T = jnp.tile(jnp.eye(16, dtype=jnp.float32), (1, 8))
  wsf = ws_ref[...]
  w128 = jnp.dot(wsf, T, preferred_element_type=jnp.float32)
  r16 = 1.0 / (dacc_ref[0] + dacc_ref[1] + wsf + 1e-16)
  r128 = jnp.dot(r16, T, preferred_element_type=jnp.float32)
  num = acc_ref[0] + acc_ref[1] + ht_ref[...] * w128
  return jnp.maximum(num * r128 + b_ref[...], 0.0)


def _mid_body(acc_ref, dacc_ref, ht_ref, ws_ref, b_ref, w_ref, as_ref, ad_ref,
              ht2_ref, s_ref, d_ref, ws2_ref):
  y = _combine(acc_ref, dacc_ref, ht_ref, ws_ref, b_ref)
  ht2 = jnp.dot(y, w_ref[...], preferred_element_type=jnp.float32)
  ht2_ref[...] = ht2
  s = jnp.dot(ht2, as_ref[...], preferred_element_type=jnp.float32)
  d = jnp.dot(ht2, ad_ref[...], preferred_element_type=jnp.float32)
  s_ref[...] = s
  d_ref[...] = d
  e = s + d
  ws2_ref[...] = jnp.exp(jnp.where(e >= 0, e, 0.2 * e))


_acc_spec = pl.BlockSpec((_NC, _BR, _HID), lambda i: (0, i, 0))
_dacc_spec = pl.BlockSpec((_NC, _BR, 16), lambda i: (0, i, 0))

_mid = pl.pallas_call(
    _mid_body,
    grid=(_NB,),
    in_specs=[_acc_spec, _dacc_spec, _row_spec, _nar_spec, _full((1, _HID)),
              _full((_HID, _HID)), _full((_HID, 16)), _full((_HID, 16))],
    out_specs=[_row_spec, _nar_spec, _nar_spec, _nar_spec],
    out_shape=[
        jax.ShapeDtypeStruct((_NP, _HID), jnp.float32),
        jax.ShapeDtypeStruct((_NP, 16), jnp.float32),
        jax.ShapeDtypeStruct((_NP, 16), jnp.float32),
        jax.ShapeDtypeStruct((_NP, 16), jnp.float32),
    ],
)


def _final_body(acc_ref, dacc_ref, ht_ref, ws_ref, b_ref, batch_ref,
                wc1_ref, bc1_ref, wc2_ref, bc2_ref, wc3_ref, bc3_ref, out_ref,
                s_acc, c_acc):
  i = pl.program_id(0)
  h2 = _combine(acc_ref, dacc_ref, ht_ref, ws_ref, b_ref)
  bt = batch_ref[0]
  Pt = (lax.broadcasted_iota(jnp.int32, (_G, _BR), 0) == bt).astype(jnp.float32)
  s = jnp.dot(Pt, h2, preferred_element_type=jnp.float32)
  cnt = jnp.dot(Pt, jnp.ones((_BR, _HID), jnp.float32),
                preferred_element_type=jnp.float32)

  @pl.when(i == 0)
  def _():
    s_acc[...] = s
    c_acc[...] = cnt

  @pl.when(i > 0)
  def _():
    s_acc[...] = s_acc[...] + s
    c_acc[...] = c_acc[...] + cnt

  @pl.when(i == _NB - 1)
  def _():
    g = s_acc[...] / jnp.maximum(c_acc[...], 1.0)
    z = jnp.maximum(jnp.dot(g, wc1_ref[...], preferred_element_type=jnp.float32)
                    + bc1_ref[...], 0.0)
    z = jnp.maximum(jnp.dot(z, wc2_ref[...], preferred_element_type=jnp.float32)
                    + bc2_ref[...], 0.0)
    out_ref[...] = jnp.dot(z, wc3_ref[...], preferred_element_type=jnp.float32) + bc3_ref[...]


_final = pl.pallas_call(
    _final_body,
    grid=(_NB,),
    in_specs=[_acc_spec, _dacc_spec, _row_spec, _nar_spec, _full((1, _HID)),
              pl.BlockSpec((1, 1, _BR), lambda i: (i, 0, 0)),
              _full((_HID, _DH)), _full((1, _DH)), _full((_DH, _DH // 2)),
              _full((1, _DH // 2)), _full((_DH // 2, _OUT)), _full((1, _OUT))],
    out_specs=pl.BlockSpec((_G, _OUT), lambda i: (0, 0)),
    out_shape=jax.ShapeDtypeStruct((_G, _OUT), jnp.float32),
    scratch_shapes=[pltpu.VMEM((_G, _HID), jnp.float32),
                    pltpu.VMEM((_G, _HID), jnp.float32)],
)


def _edge_body(src_hbm, dst_hbm, asrc_hbm, adst_hbm, ht_hbm, acc_out, dacc_out,
               acc_sp, dacc_sp, siv, div, gs, gd, gh,
               semg0, semg1, semg2, semsc0, semsc1, semsc2):
  core = lax.axis_index("c")
  sub = lax.axis_index("s")
  w = sub * _NC + core
  semg = (semg0, semg1, semg2)
  semsc = (semsc0, semsc1, semsc2)

  zero16 = jnp.zeros((16,), jnp.float32)

  def _z1(i, carry):
    for j in range(8):
      gh[i, pl.ds(j * 16, 16)] = zero16
    gs[i, :] = zero16
    return carry

  lax.fori_loop(0, _NBUF * _CH, _z1, 0)

  for off, ln in ((0, 192), (192, 192), (384, 192), (576, 56)):
    pltpu.sync_copy(gh.at[pl.ds(0, ln)], acc_sp.at[pl.ds(sub * _TPR + off, ln)])
    pltpu.sync_copy(gs.at[pl.ds(0, ln)], dacc_sp.at[pl.ds(sub * _TPR + off, ln)])
  plsc.subcore_barrier()

  def _fire_gathers(ci, b):
    return (
        pltpu.async_copy(asrc_hbm.at[siv.at[ci]], gs.at[pl.ds(b * _CH, _CH)], semg[b]),
        pltpu.async_copy(adst_hbm.at[div.at[ci]], gd.at[pl.ds(b * _CH, _CH)], semg[b]),
        pltpu.async_copy(ht_hbm.at[siv.at[ci]], gh.at[pl.ds(b * _CH, _CH)], semg[b]),
    )

  def _compute(b):
    base = b * _CH

    @plsc.parallel_loop(base, base + _CH, step=1, unroll=4)
    def _edge(c):
      e = gs[c, :] + gd[c, :]
      wv = jnp.exp(jnp.where(e >= 0, e, e * 0.2))
      gs[c, :] = wv
      for j2 in range(8):
        gh[c, pl.ds(j2 * 16, 16)] = gh[c, pl.ds(j2 * 16, 16)] * wv

  def _fire_scatters(ci, b):
    return (
        pltpu.async_copy(gh.at[pl.ds(b * _CH, _CH)], acc_sp.at[div.at[ci]], semsc[b], add=True),
        pltpu.async_copy(gs.at[pl.ds(b * _CH, _CH)], dacc_sp.at[div.at[ci]], semsc[b], add=True),
    )

  def _group(g, carry):
    row = w * _NG + g
    pltpu.sync_copy(src_hbm.at[row], siv)
    pltpu.sync_copy(dst_hbm.at[row], div)
    gcp = {0: _fire_gathers(0, 0)}
    scp = {}
    for ci in range(_GCH):
      b = ci % _NBUF
      if ci + 1 < _GCH:
        if ci - 2 >= 0:
          for cp in scp.pop(ci - 2):
            cp.wait()
        gcp[ci + 1] = _fire_gathers(ci + 1, (ci + 1) % _NBUF)
      for cp in gcp.pop(ci):
        cp.wait()
      _compute(b)
      scp[ci] = _fire_scatters(ci, b)
    for ci in (_GCH - 3, _GCH - 2, _GCH - 1):
      for cp in scp.pop(ci):
        cp.wait()
    return carry

  lax.fori_loop(0, _NG, _group, 0)
  plsc.subcore_barrier()

  pltpu.sync_copy(acc_sp.at[pl.ds(sub * _TPR, _TPR)],
                  acc_out.at[core, pl.ds(sub * _TPR, _TPR)])
  pltpu.sync_copy(dacc_sp.at[pl.ds(sub * _TPR, _TPR)],
                  dacc_out.at[core, pl.ds(sub * _TPR, _TPR)])


@functools.cache
def _make_edge_pass():
  return pl.kernel(
    _edge_body,
    out_type=[
        jax.ShapeDtypeStruct((_NC, _NP, _HID), jnp.float32),
        jax.ShapeDtypeStruct((_NC, _NP, 16), jnp.float32),
    ],
    mesh=plsc.VectorSubcoreMesh(core_axis_name="c", subcore_axis_name="s",
                                num_cores=_NC, num_subcores=_NS),
    compiler_params=pltpu.CompilerParams(use_tc_tiling_on_sc=False),
    scratch_types=[
        pltpu.VMEM_SHARED((_NP, _HID), jnp.float32),
        pltpu.VMEM_SHARED((_NP, 16), jnp.float32),
        pltpu.VMEM((_GCH, _CH), jnp.int32),
        pltpu.VMEM((_GCH, _CH), jnp.int32),
        pltpu.VMEM((_NBUF * _CH, 16), jnp.float32),
        pltpu.VMEM((_NBUF * _CH, 16), jnp.float32),
        pltpu.VMEM((_NBUF * _CH, _HID), jnp.float32),
        pltpu.SemaphoreType.DMA,
        pltpu.SemaphoreType.DMA,
        pltpu.SemaphoreType.DMA,
        pltpu.SemaphoreType.DMA,
        pltpu.SemaphoreType.DMA,
        pltpu.SemaphoreType.DMA,
    ],
  )


def _edge_pass(src3, dst3, asrc, adst, ht):
  return _make_edge_pass()(src3, dst3, asrc, adst, ht)


def kernel(x, edge_index, batch, W1, as1, ad1, b1, W2, as2, ad2, b2,
           Wc1, bc1, Wc2, bc2, Wc3, bc3):
  perm = _PERM
  xp = jnp.pad(x, ((0, _NP - _N), (0, 0)))
  pad = _N + (jnp.arange(_EP - _E, dtype=edge_index.dtype) % 16)
  src3 = jnp.concatenate([edge_index[0], pad]).reshape(_NW * _NG, _GCH, _CH)
  dst3 = jnp.concatenate([edge_index[1], pad]).reshape(_NW * _NG, _GCH, _CH)
  batch2 = jnp.pad(batch, (0, _NP - _N), constant_values=_G).reshape(_NB, 1, _BR)

  ht1, s1, d1, ws1 = _head(xp, W1[:, perm], _build_att(as1), _build_att(ad1))
  acc1, dacc1 = _edge_pass(src3, dst3, s1, d1, ht1)
  ht2, s2, d2, ws2 = _mid(acc1, dacc1, ht1, ws1, b1[perm].reshape(1, _HID),
                          W2[perm][:, perm], _build_att(as2), _build_att(ad2))
  acc2, dacc2 = _edge_pass(src3, dst3, s2, d2, ht2)
  return _final(acc2, dacc2, ht2, ws2, b2[perm].reshape(1, _HID), batch2,
                Wc1[perm], bc1.reshape(1, _DH), Wc2, bc2.reshape(1, _DH // 2),
                Wc3, bc3.reshape(1, _OUT))


# speedup vs baseline: 1.0893x; 1.0079x over previous
import functools

import jax
import jax.numpy as jnp
import numpy as np
from jax import lax
from jax.experimental import pallas as pl
from jax.experimental.pallas import tpu as pltpu
from jax.experimental.pallas import tpu_sc as plsc

_N, _E, _D, _H, _DH, _HID, _G, _OUT = 10000, 320000, 128, 8, 16, 128, 64, 10
_NP = 10112
_NC, _NS = 2, 16
_NW = _NC * _NS
_CH = 72
_NBUF = 3
_GCH = 24
_NG = 6
_CPW = _GCH * _NG
_EP = _NW * _CPW * _CH
_TPR = _NP // _NS

_PERM = np.array([h * _DH + d for d in range(_DH) for h in range(_H)], dtype=np.int32)


def _build_att(a):
  eye = jnp.eye(_H, dtype=a.dtype)
  A = jnp.einsum("hd,hj->dhj", a, eye)
  A = jnp.concatenate([A, A], axis=-1)
  return A.reshape(_HID, 2 * _H)


def _head_body(x_ref, w_ref, as_ref, ad_ref, ht_ref, s_ref, d_ref, ws_ref):
  ht = jnp.dot(x_ref[...], w_ref[...], preferred_element_type=jnp.float32)
  ht_ref[...] = ht
  s = jnp.dot(ht, as_ref[...], preferred_element_type=jnp.float32)
  d = jnp.dot(ht, ad_ref[...], preferred_element_type=jnp.float32)
  s_ref[...] = s
  d_ref[...] = d
  e = s + d
  ws_ref[...] = jnp.exp(jnp.where(e >= 0, e, 0.2 * e))


_BR = 1264
_NB = _NP // _BR

_row_spec = pl.BlockSpec((_BR, _HID), lambda i: (i, 0))
_nar_spec = pl.BlockSpec((_BR, 16), lambda i: (i, 0))
_full = lambda shape: pl.BlockSpec(shape, lambda i: tuple(0 for _ in shape))

_head = pl.pallas_call(
    _head_body,
    grid=(_NB,),
    in_specs=[_row_spec, _full((_HID, _HID)), _full((_HID, 16)), _full((_HID, 16))],
    out_specs=[_row_spec, _nar_spec, _nar_spec, _nar_spec],
    out_shape=[
        jax.ShapeDtypeStruct((_NP, _HID), jnp.float32),
        jax.ShapeDtypeStruct((_NP, 16), jnp.float32),
        jax.ShapeDtypeStruct((_NP, 16), jnp.float32),
        jax.ShapeDtypeStruct((_NP, 16), jnp.float32),
    ],
)


def _combine(acc_ref, dacc_ref, ht_ref, ws_ref, b_ref):
  T = jnp.tile(jnp.eye(16, dtype=jnp.float32), (1, 8))
  wsf = ws_ref[...]
  w128 = jnp.dot(wsf, T, preferred_element_type=jnp.float32)
  r16 = 1.0 / (dacc_ref[0] + dacc_ref[1] + wsf + 1e-16)
  r128 = jnp.dot(r16, T, preferred_element_type=jnp.float32)
  num = acc_ref[0] + acc_ref[1] + ht_ref[...] * w128
  return jnp.maximum(num * r128 + b_ref[...], 0.0)


def _mid_body(acc_ref, dacc_ref, ht_ref, ws_ref, b_ref, w_ref, as_ref, ad_ref,
              ht2_ref, s_ref, d_ref, ws2_ref):
  y = _combine(acc_ref, dacc_ref, ht_ref, ws_ref, b_ref)
  ht2 = jnp.dot(y, w_ref[...], preferred_element_type=jnp.float32)
  ht2_ref[...] = ht2
  s = jnp.dot(ht2, as_ref[...], preferred_element_type=jnp.float32)
  d = jnp.dot(ht2, ad_ref[...], preferred_element_type=jnp.float32)
  s_ref[...] = s
  d_ref[...] = d
  e = s + d
  ws2_ref[...] = jnp.exp(jnp.where(e >= 0, e, 0.2 * e))


_acc_spec = pl.BlockSpec((_NC, _BR, _HID), lambda i: (0, i, 0))
_dacc_spec = pl.BlockSpec((_NC, _BR, 16), lambda i: (0, i, 0))

_mid = pl.pallas_call(
    _mid_body,
    grid=(_NB,),
    in_specs=[_acc_spec, _dacc_spec, _row_spec, _nar_spec, _full((1, _HID)),
              _full((_HID, _HID)), _full((_HID, 16)), _full((_HID, 16))],
    out_specs=[_row_spec, _nar_spec, _nar_spec, _nar_spec],
    out_shape=[
        jax.ShapeDtypeStruct((_NP, _HID), jnp.float32),
        jax.ShapeDtypeStruct((_NP, 16), jnp.float32),
        jax.ShapeDtypeStruct((_NP, 16), jnp.float32),
        jax.ShapeDtypeStruct((_NP, 16), jnp.float32),
    ],
)


def _final_body(acc_ref, dacc_ref, ht_ref, ws_ref, b_ref, batch_ref,
                wc1_ref, bc1_ref, wc2_ref, bc2_ref, wc3_ref, bc3_ref, out_ref,
                s_acc, c_acc):
  i = pl.program_id(0)
  h2 = _combine(acc_ref, dacc_ref, ht_ref, ws_ref, b_ref)
  bt = batch_ref[0]
  Pt = (lax.broadcasted_iota(jnp.int32, (_G, _BR), 0) == bt).astype(jnp.float32)
  s = jnp.dot(Pt, h2, preferred_element_type=jnp.float32)
  cnt = jnp.dot(Pt, jnp.ones((_BR, _HID), jnp.float32),
                preferred_element_type=jnp.float32)

  @pl.when(i == 0)
  def _():
    s_acc[...] = s
    c_acc[...] = cnt

  @pl.when(i > 0)
  def _():
    s_acc[...] = s_acc[...] + s
    c_acc[...] = c_acc[...] + cnt

  @pl.when(i == _NB - 1)
  def _():
    g = s_acc[...] / jnp.maximum(c_acc[...], 1.0)
    z = jnp.maximum(jnp.dot(g, wc1_ref[...], preferred_element_type=jnp.float32)
                    + bc1_ref[...], 0.0)
    z = jnp.maximum(jnp.dot(z, wc2_ref[...], preferred_element_type=jnp.float32)
                    + bc2_ref[...], 0.0)
    out_ref[...] = jnp.dot(z, wc3_ref[...], preferred_element_type=jnp.float32) + bc3_ref[...]


_final = pl.pallas_call(
    _final_body,
    grid=(_NB,),
    in_specs=[_acc_spec, _dacc_spec, _row_spec, _nar_spec, _full((1, _HID)),
              pl.BlockSpec((1, 1, _BR), lambda i: (i, 0, 0)),
              _full((_HID, _DH)), _full((1, _DH)), _full((_DH, _DH // 2)),
              _full((1, _DH // 2)), _full((_DH // 2, _OUT)), _full((1, _OUT))],
    out_specs=pl.BlockSpec((_G, _OUT), lambda i: (0, 0)),
    out_shape=jax.ShapeDtypeStruct((_G, _OUT), jnp.float32),
    scratch_shapes=[pltpu.VMEM((_G, _HID), jnp.float32),
                    pltpu.VMEM((_G, _HID), jnp.float32)],
)


def _edge_body(src_hbm, dst_hbm, asrc_hbm, adst_hbm, ht_hbm, acc_out, dacc_out,
               acc_sp, dacc_sp, siv, div, gs, gd, gh,
               semg0, semg1, semg2, semsc0, semsc1, semsc2):
  core = lax.axis_index("c")
  sub = lax.axis_index("s")
  w = sub * _NC + core
  semg = (semg0, semg1, semg2)
  semsc = (semsc0, semsc1, semsc2)

  zero16 = jnp.zeros((16,), jnp.float32)

  def _z1(i, carry):
    for j in range(8):
      gh[i, pl.ds(j * 16, 16)] = zero16
    gs[i, :] = zero16
    return carry

  lax.fori_loop(0, _NBUF * _CH, _z1, 0)

  for off, ln in ((0, 192), (192, 192), (384, 192), (576, 56)):
    pltpu.sync_copy(gh.at[pl.ds(0, ln)], acc_sp.at[pl.ds(sub * _TPR + off, ln)])
    pltpu.sync_copy(gs.at[pl.ds(0, ln)], dacc_sp.at[pl.ds(sub * _TPR + off, ln)])
  plsc.subcore_barrier()

  def _fire_gathers(ci, b):
    return (
        pltpu.async_copy(asrc_hbm.at[siv.at[ci]], gs.at[pl.ds(b * _CH, _CH)], semg[b]),
        pltpu.async_copy(adst_hbm.at[div.at[ci]], gd.at[pl.ds(b * _CH, _CH)], semg[b]),
        pltpu.async_copy(ht_hbm.at[siv.at[ci]], gh.at[pl.ds(b * _CH, _CH)], semg[b]),
    )

  def _compute(b):
    base = b * _CH

    @plsc.parallel_loop(base, base + _CH, step=1, unroll=4)
    def _edge(c):
      e = gs[c, :] + gd[c, :]
      wv = jnp.exp(jnp.where(e >= 0, e, e * 0.2))
      gs[c, :] = wv
      for j2 in range(8):
        gh[c, pl.ds(j2 * 16, 16)] = gh[c, pl.ds(j2 * 16, 16)] * wv

  def _fire_scatters(ci, b):
    return (
        pltpu.async_copy(gh.at[pl.ds(b * _CH, _CH)], acc_sp.at[div.at[ci]], semsc[b], add=True),
        pltpu.async_copy(gs.at[pl.ds(b * _CH, _CH)], dacc_sp.at[div.at[ci]], semsc[b], add=True),
    )

  def _group(g, carry):
    row = w * _NG + g
    pltpu.sync_copy(src_hbm.at[row], siv)
    pltpu.sync_copy(dst_hbm.at[row], div)
    gcp = {0: _fire_gathers(0, 0)}
    scp = {}
    for ci in range(_GCH):
      b = ci % _NBUF
      if ci + 1 < _GCH:
        if ci - 2 >= 0:
          for cp in scp.pop(ci - 2):
            cp.wait()
        gcp[ci + 1] = _fire_gathers(ci + 1, (ci + 1) % _NBUF)
      for cp in gcp.pop(ci):
        cp.wait()
      _compute(b)
      scp[ci] = _fire_scatters(ci, b)
    for ci in (_GCH - 3, _GCH - 2, _GCH - 1):
      for cp in scp.pop(ci):
        cp.wait()
    return carry

  lax.fori_loop(0, _NG, _group, 0)
  plsc.subcore_barrier()

  pltpu.sync_copy(acc_sp.at[pl.ds(sub * _TPR, _TPR)],
                  acc_out.at[core, pl.ds(sub * _TPR, _TPR)])
  pltpu.sync_copy(dacc_sp.at[pl.ds(sub * _TPR, _TPR)],
                  dacc_out.at[core, pl.ds(sub * _TPR, _TPR)])


@functools.cache
def _make_edge_pass():
  return pl.kernel(
    _edge_body,
    out_type=[
        jax.ShapeDtypeStruct((_NC, _NP, _HID), jnp.float32),
        jax.ShapeDtypeStruct((_NC, _NP, 16), jnp.float32),
    ],
    mesh=plsc.VectorSubcoreMesh(core_axis_name="c", subcore_axis_name="s",
                                num_cores=_NC, num_subcores=_NS),
    compiler_params=pltpu.CompilerParams(use_tc_tiling_on_sc=False),
    scratch_types=[
        pltpu.VMEM_SHARED((_NP, _HID), jnp.float32),
        pltpu.VMEM_SHARED((_NP, 16), jnp.float32),
        pltpu.VMEM((_GCH, _CH), jnp.int32),
        pltpu.VMEM((_GCH, _CH), jnp.int32),
        pltpu.VMEM((_NBUF * _CH, 16), jnp.float32),
        pltpu.VMEM((_NBUF * _CH, 16), jnp.float32),
        pltpu.VMEM((_NBUF * _CH, _HID), jnp.float32),
        pltpu.SemaphoreType.DMA,
        pltpu.SemaphoreType.DMA,
        pltpu.SemaphoreType.DMA,
        pltpu.SemaphoreType.DMA,
        pltpu.SemaphoreType.DMA,
        pltpu.SemaphoreType.DMA,
    ],
  )


def _edge_pass(src3, dst3, asrc, adst, ht):
  return _make_edge_pass()(src3, dst3, asrc, adst, ht)


def kernel(x, edge_index, batch, W1, as1, ad1, b1, W2, as2, ad2, b2,
           Wc1, bc1, Wc2, bc2, Wc3, bc3):
  perm = _PERM
  xp = jnp.pad(x, ((0, _NP - _N), (0, 0)))
  pad = _N + (jnp.arange(_EP - _E, dtype=edge_index.dtype) % 16)
  src3 = jnp.concatenate([edge_index[0], pad]).reshape(_NW * _NG, _GCH, _CH)
  dst3 = jnp.concatenate([edge_index[1], pad]).reshape(_NW * _NG, _GCH, _CH)
  batch2 = jnp.pad(batch, (0, _NP - _N), constant_values=_G).reshape(_NB, 1, _BR)

  ht1, s1, d1, ws1 = _head(xp, W1[:, perm], _build_att(as1), _build_att(ad1))
  acc1, dacc1 = _edge_pass(src3, dst3, s1, d1, ht1)
  ht2, s2, d2, ws2 = _mid(acc1, dacc1, ht1, ws1, b1[perm].reshape(1, _HID),
                          W2[perm][:, perm], _build_att(as2), _build_att(ad2))
  acc2, dacc2 = _edge_pass(src3, dst3, s2, d2, ht2)
  return _final(acc2, dacc2, ht2, ws2, b2[perm].reshape(1, _HID), batch2,
                Wc1[perm], bc1.reshape(1, _DH), Wc2, bc2.reshape(1, _DH // 2),
                Wc3, bc3.reshape(1, _OUT))
